# baseline (speedup 1.0000x reference)
import functools

import jax
import jax.numpy as jnp
from jax import lax
from jax.experimental import pallas as pl
from jax.experimental.pallas import tpu as pltpu
from jax.experimental.pallas import tpu_sc as plsc

N = 10000
E = 160000
DIM = 3
K = 3
S = 2 ** DIM
KT = K ** DIM
C = 32
NCHUNK = E // C
NC = 2
NS = 16
NW = NC * NS
L = 16
ROWS_PER_TILE = N // NS
NB = 1000


def _basis_from_v(vs):
    frs, los = [], []
    for v in vs:
        ge1 = v >= 1.0
        lof = jnp.where(ge1, jnp.float32(1.0), jnp.float32(0.0))
        frs.append(v - lof)
        los.append(jnp.where(ge1, jnp.int32(1), jnp.int32(0)))
    bs_list, ws_list = [], []
    for s_ in range(S):
        bits = [(s_ >> d) & 1 for d in range(DIM)]
        b = None
        w = None
        for d in range(DIM):
            f = frs[d] if bits[d] else (1.0 - frs[d])
            b = f if b is None else b * f
            term = (los[d] + bits[d]) * (K ** d)
            w = term if w is None else w + term
        bs_list.append(b)
        ws_list.append(w)
    return bs_list, ws_list


ZROWS = 8
NUNITS = N // ZROWS


def _zero_acc(acc, zbuf, sid, width):
    zv = jnp.zeros((L,), jnp.float32)
    for j in range(ZROWS):
        for q in range(width // L):
            zbuf[j, pl.ds(q * L, L)] = zv

    @pl.loop(sid, NUNITS, step=NS)
    def _(u):
        pltpu.sync_copy(zbuf, acc.at[pl.ds(u * ZROWS, ZROWS)])


def _write_out(acc, out_hbm, cid, sid):
    @pl.loop(sid, NUNITS, step=NS)
    def _(u):
        sl = pl.ds(u * ZROWS, ZROWS)
        pltpu.sync_copy(acc.at[sl], out_hbm.at[cid, sl])


CA = 128


def _sc_stn1(pseudo_t, dst):
    mesh = plsc.VectorSubcoreMesh(core_axis_name="c", subcore_axis_name="s")
    nch = E // CA

    @functools.partial(
        pl.kernel,
        out_type=jax.ShapeDtypeStruct((NC, N, 32), jnp.float32),
        mesh=mesh,
        compiler_params=pltpu.CompilerParams(needs_layout_passes=False, use_tc_tiling_on_sc=False),
        scratch_types=[
            pltpu.VMEM_SHARED((N, 32), jnp.float32),
            pltpu.VMEM((DIM, CA), jnp.float32),
            pltpu.VMEM((CA,), jnp.int32),
            pltpu.VMEM((CA, 32), jnp.float32),
            pltpu.VMEM((ZROWS, 32), jnp.float32),
        ],
    )
    def k(p_hbm, dst_hbm, out_hbm, acc, p_v, dst_v, msg_v, zbuf):
        cid = lax.axis_index("c")
        sid = lax.axis_index("s")
        wid = sid * NC + cid

        _zero_acc(acc, zbuf, sid, 32)
        plsc.subcore_barrier()

        iota = lax.iota(jnp.int32, L)
        zv = jnp.zeros((L,), jnp.float32)
        onecol = jnp.where(iota == 11, jnp.float32(1.0), jnp.float32(0.0))

        @pl.loop(wid, nch, step=NW)
        def _(cidx):
            ebase = cidx * CA
            pltpu.sync_copy(p_hbm.at[:, pl.ds(ebase, CA)], p_v)
            pltpu.sync_copy(dst_hbm.at[pl.ds(ebase, CA)], dst_v)
            for e in range(CA):
                msg_v[e, pl.ds(0, L)] = zv
                msg_v[e, pl.ds(L, L)] = onecol
            for g in range(CA // L):
                gs = pl.ds(g * L, L)
                vs = [p_v[d, gs] * jnp.float32(K - 1) for d in range(DIM)]
                bs_list, ws_list = _basis_from_v(vs)
                rows16 = iota + g * L
                for s_ in range(S):
                    plsc.store_scatter(msg_v, [rows16, ws_list[s_]],
                                       bs_list[s_])
            pltpu.sync_copy(msg_v, acc.at[dst_v], add=True)

        plsc.subcore_barrier()
        _write_out(acc, out_hbm, cid, sid)

    return k(pseudo_t, dst)


def _sc_gather_conv(pseudo_t, src, dst, y_flat, fout, c_sz, t_pad=None):
    warp = t_pad is not None
    nq = fout // L
    nch = E // c_sz
    mesh = plsc.VectorSubcoreMesh(core_axis_name="c", subcore_axis_name="s")

    slot_scratch = [
        pltpu.VMEM((DIM, c_sz), jnp.float32),
        pltpu.VMEM((c_sz,), jnp.int32),
        pltpu.VMEM((c_sz,), jnp.int32),
        pltpu.VMEM((S * c_sz,), jnp.float32),
        pltpu.VMEM((S * c_sz,), jnp.int32),
        pltpu.VMEM((S * c_sz, fout), jnp.bfloat16),
        pltpu.SemaphoreType.DMA,
        pltpu.SemaphoreType.DMA,
        pltpu.VMEM((c_sz,), jnp.int32),
    ]
    if warp:
        slot_scratch += [
            pltpu.VMEM((c_sz, 16), jnp.float32),
            pltpu.VMEM((c_sz, 16), jnp.float32),
            pltpu.SemaphoreType.DMA,
        ]
    nslot = len(slot_scratch)

    @functools.partial(
        pl.kernel,
        out_type=jax.ShapeDtypeStruct((NC, N, fout), jnp.float32),
        mesh=mesh,
        compiler_params=pltpu.CompilerParams(
            needs_layout_passes=False, use_tc_tiling_on_sc=False),
        scratch_types=[
            pltpu.VMEM_SHARED((N, fout), jnp.float32),
            pltpu.VMEM((c_sz, fout), jnp.float32),
            pltpu.VMEM((ZROWS, fout), jnp.float32),
        ] + slot_scratch * 2,
    )
    def k(*refs):
        if warp:
            p_hbm, src_hbm, dst_hbm, t_hbm, y_hbm, out_hbm = refs[:6]
            rest = refs[6:]
        else:
            p_hbm, src_hbm, dst_hbm, y_hbm, out_hbm = refs[:5]
            rest = refs[5:]
        acc, msg_v, zbuf = rest[:3]
        slots = [rest[3:3 + nslot], rest[3 + nslot:3 + 2 * nslot]]

        cid = lax.axis_index("c")
        sid = lax.axis_index("s")
        wid = sid * NC + cid
        trips = (nch - 1 - wid) // NW + 1

        _zero_acc(acc, zbuf, sid, fout)
        plsc.subcore_barrier()

        iota = lax.iota(jnp.int32, L)

        def fire_in(b, j):
            p_v, src_v, dst_v = slots[b][:3]
            sem_in = slots[b][6]
            ebase = (wid + j * NW) * c_sz
            pltpu.async_copy(p_hbm.at[:, pl.ds(ebase, c_sz)], p_v, sem_in)
            pltpu.async_copy(src_hbm.at[pl.ds(ebase, c_sz)], src_v, sem_in)
            pltpu.async_copy(dst_hbm.at[pl.ds(ebase, c_sz)], dst_v, sem_in)

        def idx_phase(b):
            p_v, src_v, dst_v, bbuf, idx_v, rows_v, sem_in, sem_g = \
                slots[b][:8]
            pltpu.make_async_copy(p_hbm.at[:, pl.ds(0, c_sz)], p_v,
                                  sem_in).wait()
            pltpu.make_async_copy(src_hbm.at[pl.ds(0, c_sz)], src_v,
                                  sem_in).wait()
            pltpu.make_async_copy(dst_hbm.at[pl.ds(0, c_sz)], dst_v,
                                  sem_in).wait()
            if warp:
                tsrc_v, tdst_v, sem_t = slots[b][9:12]
                pltpu.async_copy(t_hbm.at[src_v], tsrc_v, sem_t)
                pltpu.async_copy(t_hbm.at[dst_v], tdst_v, sem_t)
                pltpu.make_async_copy(t_hbm.at[src_v], tsrc_v, sem_t).wait()
                pltpu.make_async_copy(t_hbm.at[dst_v], tdst_v, sem_t).wait()
            dst2_v = slots[b][8]
            for g in range(c_sz // L):
                gs = pl.ds(g * L, L)
                sv = src_v[gs]
                dst2_v[gs] = dst_v[gs]
                if warp:
                    rows16 = iota + g * L
                    vs = []
                    for d in range(DIM):
                        dvec = jnp.full((L,), d, jnp.int32)
                        ts = plsc.load_gather(tsrc_v, [rows16, dvec])
                        td = plsc.load_gather(tdst_v, [rows16, dvec])
                        npd = jnp.clip(p_v[d, gs] + td - ts,
                                       jnp.float32(0.0), jnp.float32(1.0))
                        vs.append(npd * jnp.float32(K - 1))
                else:
                    vs = [p_v[d, gs] * jnp.float32(K - 1)
                          for d in range(DIM)]
                bs_list, ws_list = _basis_from_v(vs)
                for s_ in range(S):
                    bbuf[pl.ds(s_ * c_sz + g * L, L)] = bs_list[s_]
                    idx_v[pl.ds(s_ * c_sz + g * L, L)] = \
                        ws_list[s_] * N + sv
            pltpu.async_copy(y_hbm.at[idx_v], rows_v, sem_g)

        def msg_phase(b):
            bbuf, idx_v, rows_v = slots[b][3:6]
            sem_g = slots[b][7]
            dst2_v = slots[b][8]
            pltpu.make_async_copy(y_hbm.at[idx_v], rows_v, sem_g).wait()

            nh = fout // 32

            @pl.loop(0, c_sz // 4)
            def _(e4):
                for u in range(4):
                    e = e4 * 4 + u
                    evec = jnp.full((L,), e, jnp.int32)
                    acce = [jnp.zeros((L,), jnp.float32)
                            for _ in range(nh)]
                    acco = [jnp.zeros((L,), jnp.float32)
                            for _ in range(nh)]
                    for s_ in range(S):
                        b_ = plsc.load_gather(bbuf, [evec + s_ * c_sz])
                        for h in range(nh):
                            row32 = rows_v[s_ * c_sz + e,
                                           pl.ds(h * 32, 32)]
                            ev, od = plsc.unpack(
                                row32, format=plsc.PackFormat.INTERLEAVED)
                            acce[h] = acce[h] + b_ * ev
                            acco[h] = acco[h] + b_ * od
                    for h in range(nh):
                        cbase = 32 * h + 2 * iota
                        plsc.store_scatter(msg_v, [evec, cbase], acce[h])
                        plsc.store_scatter(msg_v, [evec, cbase + 1],
                                           acco[h])

            pltpu.sync_copy(msg_v, acc.at[dst2_v], add=True)

        fire_in(0, 0)
        npairs = (trips + 1) // 2

        @pl.loop(0, npairs)
        def _(pp):
            j1 = 2 * pp + 1
            idx_phase(0)

            @pl.when(j1 < trips)
            def _():
                fire_in(1, j1)

            @pl.when(pp > 0)
            def _():
                msg_phase(1)

            @pl.when(j1 < trips)
            def _():
                idx_phase(1)

                @pl.when(j1 + 1 < trips)
                def _():
                    fire_in(0, j1 + 1)

            msg_phase(0)

        @pl.when(lax.rem(trips, 2) == 0)
        def _():
            msg_phase(1)

        plsc.subcore_barrier()
        _write_out(acc, out_hbm, cid, sid)

    if warp:
        return k(pseudo_t, src, dst, t_pad, y_flat)
    return k(pseudo_t, src, dst, y_flat)


def _elu(x):
    return jnp.where(x > 0, x, jnp.exp(x) - 1.0)


def _tc_h1(agg1, w1pad, root1, b1):
    def body(a_ref, w_ref, r_ref, b_ref, h_ref, inv_ref):
        asum = a_ref[0] + a_ref[1]
        deg = asum[:, 27:28]
        inv = 1.0 / jnp.maximum(deg, 1.0)
        inv_ref[...] = inv
        h_ref[...] = _elu(jnp.dot(asum, w_ref[...],
                                  preferred_element_type=jnp.float32) * inv
                          + r_ref[...] + b_ref[...])

    return pl.pallas_call(
        body,
        grid=(N // NB,),
        in_specs=[
            pl.BlockSpec((NC, NB, 32), lambda n: (0, n, 0)),
            pl.BlockSpec((32, 64), lambda n: (0, 0)),
            pl.BlockSpec((1, 64), lambda n: (0, 0)),
            pl.BlockSpec((64,), lambda n: (0,)),
        ],
        out_specs=[
            pl.BlockSpec((NB, 64), lambda n: (n, 0)),
            pl.BlockSpec((NB, 1), lambda n: (n, 0)),
        ],
        out_shape=[
            jax.ShapeDtypeStruct((N, 64), jnp.float32),
            jax.ShapeDtypeStruct((N, 1), jnp.float32),
        ],
    )(agg1, w1pad, root1, b1)


def _tc_table(x, w, out_dtype):
    kt, fin, fo = w.shape

    def body(x_ref, w_ref, o_ref):
        o_ref[...] = jnp.dot(x_ref[...], w_ref[0],
                             preferred_element_type=jnp.float32
                             ).astype(out_dtype)[None]

    return pl.pallas_call(
        body,
        grid=(N // NB, kt),
        in_specs=[
            pl.BlockSpec((NB, fin), lambda n, k_: (n, 0)),
            pl.BlockSpec((1, fin, fo), lambda n, k_: (k_, 0, 0)),
        ],
        out_specs=pl.BlockSpec((1, NB, fo), lambda n, k_: (k_, n, 0)),
        out_shape=jax.ShapeDtypeStruct((kt, N, fo), out_dtype),
    )(x, w)


def _tc_t(agg2, inv_deg, h1, root2, b2, w3, b3, w4p, b4p):
    def body(a_ref, i_ref, h_ref, r_ref, b2_ref, w3_ref, b3_ref,
             w4_ref, b4_ref, t_ref):
        h2 = _elu((a_ref[0] + a_ref[1]) * i_ref[...]
                  + jnp.dot(h_ref[...], r_ref[...],
                            preferred_element_type=jnp.float32)
                  + b2_ref[...])
        h3 = _elu(jnp.dot(h2, w3_ref[...],
                          preferred_element_type=jnp.float32) + b3_ref[...])
        t_ref[...] = jnp.dot(h3, w4_ref[...],
                             preferred_element_type=jnp.float32) + b4_ref[...]

    return pl.pallas_call(
        body,
        grid=(N // NB,),
        in_specs=[
            pl.BlockSpec((NC, NB, 64), lambda n: (0, n, 0)),
            pl.BlockSpec((NB, 1), lambda n: (n, 0)),
            pl.BlockSpec((NB, 64), lambda n: (n, 0)),
            pl.BlockSpec((64, 64), lambda n: (0, 0)),
            pl.BlockSpec((64,), lambda n: (0,)),
            pl.BlockSpec((64, 64), lambda n: (0, 0)),
            pl.BlockSpec((64,), lambda n: (0,)),
            pl.BlockSpec((64, 16), lambda n: (0, 0)),
            pl.BlockSpec((16,), lambda n: (0,)),
        ],
        out_specs=pl.BlockSpec((NB, 16), lambda n: (n, 0)),
        out_shape=jax.ShapeDtypeStruct((N, 16), jnp.float32),
    )(agg2, inv_deg, h1, root2, b2, w3, b3, w4p, b4p)


def _tc_final(agg3, inv_deg, x, root_w, bias):
    def body(a_ref, i_ref, x_ref, r_ref, b_ref, o_ref):
        o_ref[...] = ((a_ref[0] + a_ref[1]) * i_ref[...]
                      + jnp.dot(x_ref[...], r_ref[...],
                                preferred_element_type=jnp.float32)
                      + b_ref[...])

    return pl.pallas_call(
        body,
        grid=(N // NB,),
        in_specs=[
            pl.BlockSpec((NC, NB, 128), lambda n: (0, n, 0)),
            pl.BlockSpec((NB, 1), lambda n: (n, 0)),
            pl.BlockSpec((NB, 128), lambda n: (n, 0)),
            pl.BlockSpec((128, 128), lambda n: (0, 0)),
            pl.BlockSpec((128,), lambda n: (0,)),
        ],
        out_specs=pl.BlockSpec((NB, 128), lambda n: (n, 0)),
        out_shape=jax.ShapeDtypeStruct((N, 128), jnp.float32),
    )(agg3, inv_deg, x, root_w, bias)


def kernel(input, edge_index, pseudo, stn1_w, stn1_root, stn1_b,
           stn2_w, stn2_root, stn2_b, stn3_w, stn3_b, stn4_w, stn4_b,
           conv_w, conv_root, conv_b):
    src = edge_index[0]
    dst = edge_index[1]
    pseudo_t = pseudo.T

    agg1 = _sc_stn1(pseudo_t, dst)
    w1pad = jnp.zeros((32, 64), jnp.float32).at[:KT].set(
        stn1_w.reshape(KT, 64))
    h1, inv_deg = _tc_h1(agg1, w1pad, stn1_root, stn1_b)

    y2 = _tc_table(h1, stn2_w, jnp.bfloat16).reshape(KT * N, 64)
    agg2 = _sc_gather_conv(pseudo_t, src, dst, y2, 64, 32)

    w4p = jnp.zeros((64, 16), jnp.float32).at[:, :DIM].set(stn4_w)
    b4p = jnp.zeros((16,), jnp.float32).at[:DIM].set(stn4_b)
    t_pad = _tc_t(agg2, inv_deg, h1, stn2_root, stn2_b,
                  stn3_w, stn3_b, w4p, b4p)

    y3 = _tc_table(input, conv_w, jnp.bfloat16).reshape(KT * N, 128)
    agg3 = _sc_gather_conv(pseudo_t, src, dst, y3, 128, 32, t_pad=t_pad)

    return _tc_final(agg3, inv_deg, input, conv_root, conv_b)


# speedup vs baseline: 1.0287x; 1.0287x over previous
import functools

import jax
import jax.numpy as jnp
from jax import lax
from jax.experimental import pallas as pl
from jax.experimental.pallas import tpu as pltpu
from jax.experimental.pallas import tpu_sc as plsc

N = 10000
E = 160000
DIM = 3
K = 3
S = 2 ** DIM
KT = K ** DIM
C = 32
NCHUNK = E // C
NC = 2
NS = 16
NW = NC * NS
L = 16
ROWS_PER_TILE = N // NS
NB = 1000


def _basis_from_v(vs):
    frs, los = [], []
    for v in vs:
        ge1 = v >= 1.0
        lof = jnp.where(ge1, jnp.float32(1.0), jnp.float32(0.0))
        frs.append(v - lof)
        los.append(jnp.where(ge1, jnp.int32(1), jnp.int32(0)))
    bs_list, ws_list = [], []
    for s_ in range(S):
        bits = [(s_ >> d) & 1 for d in range(DIM)]
        b = None
        w = None
        for d in range(DIM):
            f = frs[d] if bits[d] else (1.0 - frs[d])
            b = f if b is None else b * f
            term = (los[d] + bits[d]) * (K ** d)
            w = term if w is None else w + term
        bs_list.append(b)
        ws_list.append(w)
    return bs_list, ws_list


ZROWS = 8
NUNITS = N // ZROWS


def _zero_acc(acc, zbuf, sid, width):
    zv = jnp.zeros((L,), jnp.float32)
    for j in range(ZROWS):
        for q in range(width // L):
            zbuf[j, pl.ds(q * L, L)] = zv

    @pl.loop(sid, NUNITS, step=NS)
    def _(u):
        pltpu.sync_copy(zbuf, acc.at[pl.ds(u * ZROWS, ZROWS)])


def _write_out(acc, out_hbm, cid, sid):
    @pl.loop(sid, NUNITS, step=NS)
    def _(u):
        sl = pl.ds(u * ZROWS, ZROWS)
        pltpu.sync_copy(acc.at[sl], out_hbm.at[cid, sl])


CA = 128


def _sc_stn1(pseudo_t, dst):
    mesh = plsc.VectorSubcoreMesh(core_axis_name="c", subcore_axis_name="s")
    nch = E // CA

    @functools.partial(
        pl.kernel,
        out_type=jax.ShapeDtypeStruct((NC, N, 32), jnp.float32),
        mesh=mesh,
        compiler_params=pltpu.CompilerParams(needs_layout_passes=False, use_tc_tiling_on_sc=False),
        scratch_types=[
            pltpu.VMEM_SHARED((N, 32), jnp.float32),
            pltpu.VMEM((DIM, CA), jnp.float32),
            pltpu.VMEM((CA,), jnp.int32),
            pltpu.VMEM((CA, 32), jnp.float32),
            pltpu.VMEM((ZROWS, 32), jnp.float32),
        ],
    )
    def k(p_hbm, dst_hbm, out_hbm, acc, p_v, dst_v, msg_v, zbuf):
        cid = lax.axis_index("c")
        sid = lax.axis_index("s")
        wid = sid * NC + cid

        _zero_acc(acc, zbuf, sid, 32)
        plsc.subcore_barrier()

        iota = lax.iota(jnp.int32, L)
        zv = jnp.zeros((L,), jnp.float32)
        onecol = jnp.where(iota == 11, jnp.float32(1.0), jnp.float32(0.0))

        @pl.loop(wid, nch, step=NW)
        def _(cidx):
            ebase = cidx * CA
            pltpu.sync_copy(p_hbm.at[:, pl.ds(ebase, CA)], p_v)
            pltpu.sync_copy(dst_hbm.at[pl.ds(ebase, CA)], dst_v)
            for e in range(CA):
                msg_v[e, pl.ds(0, L)] = zv
                msg_v[e, pl.ds(L, L)] = onecol
            for g in range(CA // L):
                gs = pl.ds(g * L, L)
                vs = [p_v[d, gs] * jnp.float32(K - 1) for d in range(DIM)]
                bs_list, ws_list = _basis_from_v(vs)
                rows16 = iota + g * L
                for s_ in range(S):
                    plsc.store_scatter(msg_v, [rows16, ws_list[s_]],
                                       bs_list[s_])
            pltpu.sync_copy(msg_v, acc.at[dst_v], add=True)

        plsc.subcore_barrier()
        _write_out(acc, out_hbm, cid, sid)

    return k(pseudo_t, dst)


def _sc_gather_conv(pseudo_t, src, dst, y_flat, fout, c_sz, t_pad=None):
    warp = t_pad is not None
    nq = fout // L
    nch = E // c_sz
    mesh = plsc.VectorSubcoreMesh(core_axis_name="c", subcore_axis_name="s")

    slot_scratch = [
        pltpu.VMEM((DIM, c_sz), jnp.float32),
        pltpu.VMEM((c_sz,), jnp.int32),
        pltpu.VMEM((c_sz,), jnp.int32),
        pltpu.VMEM((S * c_sz,), jnp.float32),
        pltpu.VMEM((S * c_sz,), jnp.int32),
        pltpu.VMEM((S * c_sz, fout), jnp.bfloat16),
        pltpu.SemaphoreType.DMA,
        pltpu.SemaphoreType.DMA,
        pltpu.VMEM((c_sz,), jnp.int32),
    ]
    if warp:
        slot_scratch += [
            pltpu.VMEM((c_sz, 16), jnp.float32),
            pltpu.VMEM((c_sz, 16), jnp.float32),
            pltpu.SemaphoreType.DMA,
        ]
    nslot = len(slot_scratch)

    @functools.partial(
        pl.kernel,
        out_type=jax.ShapeDtypeStruct((NC, N, fout), jnp.float32),
        mesh=mesh,
        compiler_params=pltpu.CompilerParams(
            needs_layout_passes=False, use_tc_tiling_on_sc=False),
        scratch_types=[
            pltpu.VMEM_SHARED((N, fout), jnp.float32),
            pltpu.VMEM((c_sz, fout), jnp.float32),
            pltpu.VMEM((ZROWS, fout), jnp.float32),
        ] + slot_scratch * 2,
    )
    def k(*refs):
        if warp:
            p_hbm, src_hbm, dst_hbm, t_hbm, y_hbm, out_hbm = refs[:6]
            rest = refs[6:]
        else:
            p_hbm, src_hbm, dst_hbm, y_hbm, out_hbm = refs[:5]
            rest = refs[5:]
        acc, msg_v, zbuf = rest[:3]
        slots = [rest[3:3 + nslot], rest[3 + nslot:3 + 2 * nslot]]

        cid = lax.axis_index("c")
        sid = lax.axis_index("s")
        wid = sid * NC + cid
        trips = (nch - 1 - wid) // NW + 1

        _zero_acc(acc, zbuf, sid, fout)
        plsc.subcore_barrier()

        iota = lax.iota(jnp.int32, L)

        def fire_in(b, j):
            p_v, src_v, dst_v = slots[b][:3]
            sem_in = slots[b][6]
            ebase = (wid + j * NW) * c_sz
            pltpu.async_copy(p_hbm.at[:, pl.ds(ebase, c_sz)], p_v, sem_in)
            pltpu.async_copy(src_hbm.at[pl.ds(ebase, c_sz)], src_v, sem_in)
            pltpu.async_copy(dst_hbm.at[pl.ds(ebase, c_sz)], dst_v, sem_in)

        def idx_phase(b):
            p_v, src_v, dst_v, bbuf, idx_v, rows_v, sem_in, sem_g = \
                slots[b][:8]
            pltpu.make_async_copy(p_hbm.at[:, pl.ds(0, c_sz)], p_v,
                                  sem_in).wait()
            pltpu.make_async_copy(src_hbm.at[pl.ds(0, c_sz)], src_v,
                                  sem_in).wait()
            pltpu.make_async_copy(dst_hbm.at[pl.ds(0, c_sz)], dst_v,
                                  sem_in).wait()
            if warp:
                tsrc_v, tdst_v, sem_t = slots[b][9:12]
                pltpu.async_copy(t_hbm.at[src_v], tsrc_v, sem_t)
                pltpu.async_copy(t_hbm.at[dst_v], tdst_v, sem_t)
                pltpu.make_async_copy(t_hbm.at[src_v], tsrc_v, sem_t).wait()
                pltpu.make_async_copy(t_hbm.at[dst_v], tdst_v, sem_t).wait()
            dst2_v = slots[b][8]
            for g in range(c_sz // L):
                gs = pl.ds(g * L, L)
                sv = src_v[gs]
                dst2_v[gs] = dst_v[gs]
                if warp:
                    rows16 = iota + g * L
                    vs = []
                    for d in range(DIM):
                        dvec = jnp.full((L,), d, jnp.int32)
                        ts = plsc.load_gather(tsrc_v, [rows16, dvec])
                        td = plsc.load_gather(tdst_v, [rows16, dvec])
                        npd = jnp.clip(p_v[d, gs] + td - ts,
                                       jnp.float32(0.0), jnp.float32(1.0))
                        vs.append(npd * jnp.float32(K - 1))
                else:
                    vs = [p_v[d, gs] * jnp.float32(K - 1)
                          for d in range(DIM)]
                bs_list, ws_list = _basis_from_v(vs)
                for s_ in range(S):
                    bbuf[pl.ds(s_ * c_sz + g * L, L)] = bs_list[s_]
                    idx_v[pl.ds(s_ * c_sz + g * L, L)] = \
                        ws_list[s_] * N + sv
            pltpu.async_copy(y_hbm.at[idx_v], rows_v, sem_g)

        def msg_phase(b):
            bbuf, idx_v, rows_v = slots[b][3:6]
            sem_g = slots[b][7]
            dst2_v = slots[b][8]
            pltpu.make_async_copy(y_hbm.at[idx_v], rows_v, sem_g).wait()

            nh = fout // 32

            @pl.loop(0, c_sz // 4)
            def _(e4):
                for u in range(4):
                    e = e4 * 4 + u
                    evec = jnp.full((L,), e, jnp.int32)
                    acce = [jnp.zeros((L,), jnp.float32)
                            for _ in range(nh)]
                    acco = [jnp.zeros((L,), jnp.float32)
                            for _ in range(nh)]
                    for s_ in range(S):
                        b_ = plsc.load_gather(bbuf, [evec + s_ * c_sz])
                        for h in range(nh):
                            row32 = rows_v[s_ * c_sz + e,
                                           pl.ds(h * 32, 32)]
                            ev, od = plsc.unpack(
                                row32, format=plsc.PackFormat.INTERLEAVED)
                            acce[h] = acce[h] + b_ * ev
                            acco[h] = acco[h] + b_ * od
                    for h in range(nh):
                        cbase = 32 * h + 2 * iota
                        plsc.store_scatter(msg_v, [evec, cbase], acce[h])
                        plsc.store_scatter(msg_v, [evec, cbase + 1],
                                           acco[h])

            pltpu.sync_copy(msg_v, acc.at[dst2_v], add=True)

        fire_in(0, 0)
        npairs = (trips + 1) // 2

        @pl.loop(0, npairs)
        def _(pp):
            j1 = 2 * pp + 1
            idx_phase(0)

            @pl.when(j1 < trips)
            def _():
                fire_in(1, j1)

            @pl.when(pp > 0)
            def _():
                msg_phase(1)

            @pl.when(j1 < trips)
            def _():
                idx_phase(1)

                @pl.when(j1 + 1 < trips)
                def _():
                    fire_in(0, j1 + 1)

            msg_phase(0)

        @pl.when(lax.rem(trips, 2) == 0)
        def _():
            msg_phase(1)

        plsc.subcore_barrier()
        _write_out(acc, out_hbm, cid, sid)

    if warp:
        return k(pseudo_t, src, dst, t_pad, y_flat)
    return k(pseudo_t, src, dst, y_flat)


def _elu(x):
    return jnp.where(x > 0, x, jnp.exp(x) - 1.0)


def _tc_h1(agg1, w1pad, root1, b1):
    def body(a_ref, w_ref, r_ref, b_ref, h_ref, inv_ref):
        asum = a_ref[0] + a_ref[1]
        deg = asum[:, 27:28]
        inv = 1.0 / jnp.maximum(deg, 1.0)
        inv_ref[...] = inv
        h_ref[...] = _elu(jnp.dot(asum, w_ref[...],
                                  preferred_element_type=jnp.float32) * inv
                          + r_ref[...] + b_ref[...])

    return pl.pallas_call(
        body,
        grid=(N // NB,),
        in_specs=[
            pl.BlockSpec((NC, NB, 32), lambda n: (0, n, 0)),
            pl.BlockSpec((32, 64), lambda n: (0, 0)),
            pl.BlockSpec((1, 64), lambda n: (0, 0)),
            pl.BlockSpec((64,), lambda n: (0,)),
        ],
        out_specs=[
            pl.BlockSpec((NB, 64), lambda n: (n, 0)),
            pl.BlockSpec((NB, 1), lambda n: (n, 0)),
        ],
        out_shape=[
            jax.ShapeDtypeStruct((N, 64), jnp.float32),
            jax.ShapeDtypeStruct((N, 1), jnp.float32),
        ],
    )(agg1, w1pad, root1, b1)


def _tc_table(x, w, out_dtype):
    kt, fin, fo = w.shape

    nb_per = N // NB

    def body(x_ref, w_ref, o_ref):
        o_ref[...] = jnp.dot(x_ref[...], w_ref[0],
                             preferred_element_type=jnp.float32
                             ).astype(out_dtype)

    return pl.pallas_call(
        body,
        grid=(nb_per, kt),
        in_specs=[
            pl.BlockSpec((NB, fin), lambda n, k_: (n, 0)),
            pl.BlockSpec((1, fin, fo), lambda n, k_: (k_, 0, 0)),
        ],
        out_specs=pl.BlockSpec((NB, fo), lambda n, k_: (k_ * nb_per + n, 0)),
        out_shape=jax.ShapeDtypeStruct((kt * N, fo), out_dtype),
    )(x, w)


def _tc_t(agg2, inv_deg, h1, root2, b2, w3, b3, w4p, b4p):
    def body(a_ref, i_ref, h_ref, r_ref, b2_ref, w3_ref, b3_ref,
             w4_ref, b4_ref, t_ref):
        h2 = _elu((a_ref[0] + a_ref[1]) * i_ref[...]
                  + jnp.dot(h_ref[...], r_ref[...],
                            preferred_element_type=jnp.float32)
                  + b2_ref[...])
        h3 = _elu(jnp.dot(h2, w3_ref[...],
                          preferred_element_type=jnp.float32) + b3_ref[...])
        t_ref[...] = jnp.dot(h3, w4_ref[...],
                             preferred_element_type=jnp.float32) + b4_ref[...]

    return pl.pallas_call(
        body,
        grid=(N // NB,),
        in_specs=[
            pl.BlockSpec((NC, NB, 64), lambda n: (0, n, 0)),
            pl.BlockSpec((NB, 1), lambda n: (n, 0)),
            pl.BlockSpec((NB, 64), lambda n: (n, 0)),
            pl.BlockSpec((64, 64), lambda n: (0, 0)),
            pl.BlockSpec((64,), lambda n: (0,)),
            pl.BlockSpec((64, 64), lambda n: (0, 0)),
            pl.BlockSpec((64,), lambda n: (0,)),
            pl.BlockSpec((64, 16), lambda n: (0, 0)),
            pl.BlockSpec((16,), lambda n: (0,)),
        ],
        out_specs=pl.BlockSpec((NB, 16), lambda n: (n, 0)),
        out_shape=jax.ShapeDtypeStruct((N, 16), jnp.float32),
    )(agg2, inv_deg, h1, root2, b2, w3, b3, w4p, b4p)


def _tc_final(agg3, inv_deg, x, root_w, bias):
    def body(a_ref, i_ref, x_ref, r_ref, b_ref, o_ref):
        o_ref[...] = ((a_ref[0] + a_ref[1]) * i_ref[...]
                      + jnp.dot(x_ref[...], r_ref[...],
                                preferred_element_type=jnp.float32)
                      + b_ref[...])

    return pl.pallas_call(
        body,
        grid=(N // NB,),
        in_specs=[
            pl.BlockSpec((NC, NB, 128), lambda n: (0, n, 0)),
            pl.BlockSpec((NB, 1), lambda n: (n, 0)),
            pl.BlockSpec((NB, 128), lambda n: (n, 0)),
            pl.BlockSpec((128, 128), lambda n: (0, 0)),
            pl.BlockSpec((128,), lambda n: (0,)),
        ],
        out_specs=pl.BlockSpec((NB, 128), lambda n: (n, 0)),
        out_shape=jax.ShapeDtypeStruct((N, 128), jnp.float32),
    )(agg3, inv_deg, x, root_w, bias)


def kernel(input, edge_index, pseudo, stn1_w, stn1_root, stn1_b,
           stn2_w, stn2_root, stn2_b, stn3_w, stn3_b, stn4_w, stn4_b,
           conv_w, conv_root, conv_b):
    src = edge_index[0]
    dst = edge_index[1]
    pseudo_t = pseudo.T

    agg1 = _sc_stn1(pseudo_t, dst)
    w1pad = jnp.zeros((32, 64), jnp.float32).at[:KT].set(
        stn1_w.reshape(KT, 64))
    h1, inv_deg = _tc_h1(agg1, w1pad, stn1_root, stn1_b)

    y2 = _tc_table(h1, stn2_w, jnp.bfloat16)
    agg2 = _sc_gather_conv(pseudo_t, src, dst, y2, 64, 32)

    w4p = jnp.zeros((64, 16), jnp.float32).at[:, :DIM].set(stn4_w)
    b4p = jnp.zeros((16,), jnp.float32).at[:DIM].set(stn4_b)
    t_pad = _tc_t(agg2, inv_deg, h1, stn2_root, stn2_b,
                  stn3_w, stn3_b, w4p, b4p)

    y3 = _tc_table(input, conv_w, jnp.bfloat16)
    agg3 = _sc_gather_conv(pseudo_t, src, dst, y3, 128, 32, t_pad=t_pad)

    return _tc_final(agg3, inv_deg, input, conv_root, conv_b)


# speedup vs baseline: 1.0437x; 1.0146x over previous
import functools

import jax
import jax.numpy as jnp
from jax import lax
from jax.experimental import pallas as pl
from jax.experimental.pallas import tpu as pltpu
from jax.experimental.pallas import tpu_sc as plsc

N = 10000
E = 160000
DIM = 3
K = 3
S = 2 ** DIM
KT = K ** DIM
C = 32
NCHUNK = E // C
NC = 2
NS = 16
NW = NC * NS
L = 16
ROWS_PER_TILE = N // NS
NB = 1000


def _basis_from_v(vs):
    frs, los = [], []
    for v in vs:
        ge1 = v >= 1.0
        lof = jnp.where(ge1, jnp.float32(1.0), jnp.float32(0.0))
        frs.append(v - lof)
        los.append(jnp.where(ge1, jnp.int32(1), jnp.int32(0)))
    bs_list, ws_list = [], []
    for s_ in range(S):
        bits = [(s_ >> d) & 1 for d in range(DIM)]
        b = None
        w = None
        for d in range(DIM):
            f = frs[d] if bits[d] else (1.0 - frs[d])
            b = f if b is None else b * f
            term = (los[d] + bits[d]) * (K ** d)
            w = term if w is None else w + term
        bs_list.append(b)
        ws_list.append(w)
    return bs_list, ws_list


ZROWS = 8
NUNITS = N // ZROWS


def _zero_acc(acc, zbuf, sid, width):
    zv = jnp.zeros((L,), jnp.float32)
    for j in range(ZROWS):
        for q in range(width // L):
            zbuf[j, pl.ds(q * L, L)] = zv

    @pl.loop(sid, NUNITS, step=NS)
    def _(u):
        pltpu.sync_copy(zbuf, acc.at[pl.ds(u * ZROWS, ZROWS)])


def _write_out(acc, out_hbm, cid, sid):
    @pl.loop(sid, NUNITS, step=NS)
    def _(u):
        sl = pl.ds(u * ZROWS, ZROWS)
        pltpu.sync_copy(acc.at[sl], out_hbm.at[cid, sl])


CA = 128


def _sc_stn1(pseudo_t, dst):
    mesh = plsc.VectorSubcoreMesh(core_axis_name="c", subcore_axis_name="s")
    nch = E // CA

    @functools.partial(
        pl.kernel,
        out_type=jax.ShapeDtypeStruct((NC, N, 32), jnp.float32),
        mesh=mesh,
        compiler_params=pltpu.CompilerParams(needs_layout_passes=False, use_tc_tiling_on_sc=False),
        scratch_types=[
            pltpu.VMEM_SHARED((N, 32), jnp.float32),
            pltpu.VMEM((DIM, CA), jnp.float32),
            pltpu.VMEM((CA,), jnp.int32),
            pltpu.VMEM((CA, 32), jnp.float32),
            pltpu.VMEM((ZROWS, 32), jnp.float32),
        ],
    )
    def k(p_hbm, dst_hbm, out_hbm, acc, p_v, dst_v, msg_v, zbuf):
        cid = lax.axis_index("c")
        sid = lax.axis_index("s")
        wid = sid * NC + cid

        _zero_acc(acc, zbuf, sid, 32)
        plsc.subcore_barrier()

        iota = lax.iota(jnp.int32, L)
        zv = jnp.zeros((L,), jnp.float32)
        onecol = jnp.where(iota == 11, jnp.float32(1.0), jnp.float32(0.0))

        @pl.loop(wid, nch, step=NW)
        def _(cidx):
            ebase = cidx * CA
            pltpu.sync_copy(p_hbm.at[:, pl.ds(ebase, CA)], p_v)
            pltpu.sync_copy(dst_hbm.at[pl.ds(ebase, CA)], dst_v)
            for e in range(CA):
                msg_v[e, pl.ds(0, L)] = zv
                msg_v[e, pl.ds(L, L)] = onecol
            for g in range(CA // L):
                gs = pl.ds(g * L, L)
                vs = [p_v[d, gs] * jnp.float32(K - 1) for d in range(DIM)]
                bs_list, ws_list = _basis_from_v(vs)
                rows16 = iota + g * L
                for s_ in range(S):
                    plsc.store_scatter(msg_v, [rows16, ws_list[s_]],
                                       bs_list[s_])
            pltpu.sync_copy(msg_v, acc.at[dst_v], add=True)

        plsc.subcore_barrier()
        _write_out(acc, out_hbm, cid, sid)

    return k(pseudo_t, dst)


def _sc_gather_conv(pseudo_t, src, dst, y_flat, fout, c_sz, t_pad=None):
    warp = t_pad is not None
    bf16 = y_flat.dtype == jnp.bfloat16
    nq = fout // L
    nch = E // c_sz
    mesh = plsc.VectorSubcoreMesh(core_axis_name="c", subcore_axis_name="s")

    slot_scratch = [
        pltpu.VMEM((DIM, c_sz), jnp.float32),
        pltpu.VMEM((c_sz,), jnp.int32),
        pltpu.VMEM((c_sz,), jnp.int32),
        pltpu.VMEM((S * c_sz,), jnp.float32),
        pltpu.VMEM((S * c_sz,), jnp.int32),
        pltpu.VMEM((S * c_sz, fout),
                   jnp.bfloat16 if bf16 else jnp.float32),
        pltpu.SemaphoreType.DMA,
        pltpu.SemaphoreType.DMA,
        pltpu.VMEM((c_sz,), jnp.int32),
    ]
    if warp:
        slot_scratch += [
            pltpu.VMEM((c_sz, 16), jnp.float32),
            pltpu.VMEM((c_sz, 16), jnp.float32),
            pltpu.SemaphoreType.DMA,
        ]
    nslot = len(slot_scratch)

    @functools.partial(
        pl.kernel,
        out_type=jax.ShapeDtypeStruct((NC, N, fout), jnp.float32),
        mesh=mesh,
        compiler_params=pltpu.CompilerParams(
            needs_layout_passes=False, use_tc_tiling_on_sc=False),
        scratch_types=[
            pltpu.VMEM_SHARED((N, fout), jnp.float32),
            pltpu.VMEM((c_sz, fout), jnp.float32),
            pltpu.VMEM((ZROWS, fout), jnp.float32),
        ] + slot_scratch * 2,
    )
    def k(*refs):
        if warp:
            p_hbm, src_hbm, dst_hbm, t_hbm, y_hbm, out_hbm = refs[:6]
            rest = refs[6:]
        else:
            p_hbm, src_hbm, dst_hbm, y_hbm, out_hbm = refs[:5]
            rest = refs[5:]
        acc, msg_v, zbuf = rest[:3]
        slots = [rest[3:3 + nslot], rest[3 + nslot:3 + 2 * nslot]]

        cid = lax.axis_index("c")
        sid = lax.axis_index("s")
        wid = sid * NC + cid
        trips = (nch - 1 - wid) // NW + 1

        _zero_acc(acc, zbuf, sid, fout)
        plsc.subcore_barrier()

        iota = lax.iota(jnp.int32, L)

        def fire_in(b, j):
            p_v, src_v, dst_v = slots[b][:3]
            sem_in = slots[b][6]
            ebase = (wid + j * NW) * c_sz
            pltpu.async_copy(p_hbm.at[:, pl.ds(ebase, c_sz)], p_v, sem_in)
            pltpu.async_copy(src_hbm.at[pl.ds(ebase, c_sz)], src_v, sem_in)
            pltpu.async_copy(dst_hbm.at[pl.ds(ebase, c_sz)], dst_v, sem_in)

        def idx_phase(b):
            p_v, src_v, dst_v, bbuf, idx_v, rows_v, sem_in, sem_g = \
                slots[b][:8]
            pltpu.make_async_copy(p_hbm.at[:, pl.ds(0, c_sz)], p_v,
                                  sem_in).wait()
            pltpu.make_async_copy(src_hbm.at[pl.ds(0, c_sz)], src_v,
                                  sem_in).wait()
            pltpu.make_async_copy(dst_hbm.at[pl.ds(0, c_sz)], dst_v,
                                  sem_in).wait()
            if warp:
                tsrc_v, tdst_v, sem_t = slots[b][9:12]
                pltpu.async_copy(t_hbm.at[src_v], tsrc_v, sem_t)
                pltpu.async_copy(t_hbm.at[dst_v], tdst_v, sem_t)
                pltpu.make_async_copy(t_hbm.at[src_v], tsrc_v, sem_t).wait()
                pltpu.make_async_copy(t_hbm.at[dst_v], tdst_v, sem_t).wait()
            dst2_v = slots[b][8]
            for g in range(c_sz // L):
                gs = pl.ds(g * L, L)
                sv = src_v[gs]
                dst2_v[gs] = dst_v[gs]
                if warp:
                    rows16 = iota + g * L
                    vs = []
                    for d in range(DIM):
                        dvec = jnp.full((L,), d, jnp.int32)
                        ts = plsc.load_gather(tsrc_v, [rows16, dvec])
                        td = plsc.load_gather(tdst_v, [rows16, dvec])
                        npd = jnp.clip(p_v[d, gs] + td - ts,
                                       jnp.float32(0.0), jnp.float32(1.0))
                        vs.append(npd * jnp.float32(K - 1))
                else:
                    vs = [p_v[d, gs] * jnp.float32(K - 1)
                          for d in range(DIM)]
                bs_list, ws_list = _basis_from_v(vs)
                for s_ in range(S):
                    bbuf[pl.ds(s_ * c_sz + g * L, L)] = bs_list[s_]
                    idx_v[pl.ds(s_ * c_sz + g * L, L)] = \
                        ws_list[s_] * N + sv
            pltpu.async_copy(y_hbm.at[idx_v], rows_v, sem_g)

        def msg_phase(b):
            bbuf, idx_v, rows_v = slots[b][3:6]
            sem_g = slots[b][7]
            dst2_v = slots[b][8]
            pltpu.make_async_copy(y_hbm.at[idx_v], rows_v, sem_g).wait()

            nh = fout // 32

            @pl.loop(0, c_sz // 4)
            def _(e4):
                for u in range(4):
                    e = e4 * 4 + u
                    evec = jnp.full((L,), e, jnp.int32)
                    if bf16:
                        acce = [jnp.zeros((L,), jnp.float32)
                                for _ in range(nh)]
                        acco = [jnp.zeros((L,), jnp.float32)
                                for _ in range(nh)]
                        for s_ in range(S):
                            b_ = plsc.load_gather(bbuf,
                                                  [evec + s_ * c_sz])
                            for h in range(nh):
                                row32 = rows_v[s_ * c_sz + e,
                                               pl.ds(h * 32, 32)]
                                ev, od = plsc.unpack(
                                    row32,
                                    format=plsc.PackFormat.INTERLEAVED)
                                acce[h] = acce[h] + b_ * ev
                                acco[h] = acco[h] + b_ * od
                        for h in range(nh):
                            cbase = 32 * h + 2 * iota
                            plsc.store_scatter(msg_v, [evec, cbase],
                                               acce[h])
                            plsc.store_scatter(msg_v, [evec, cbase + 1],
                                               acco[h])
                    else:
                        accs = [jnp.zeros((L,), jnp.float32)
                                for _ in range(nq)]
                        for s_ in range(S):
                            b_ = plsc.load_gather(bbuf,
                                                  [evec + s_ * c_sz])
                            for q in range(nq):
                                row = rows_v[s_ * c_sz + e,
                                             pl.ds(q * L, L)]
                                accs[q] = accs[q] + b_ * row
                        for q in range(nq):
                            msg_v[e, pl.ds(q * L, L)] = accs[q]

            pltpu.sync_copy(msg_v, acc.at[dst2_v], add=True)

        fire_in(0, 0)
        npairs = (trips + 1) // 2

        @pl.loop(0, npairs)
        def _(pp):
            j1 = 2 * pp + 1
            idx_phase(0)

            @pl.when(j1 < trips)
            def _():
                fire_in(1, j1)

            @pl.when(pp > 0)
            def _():
                msg_phase(1)

            @pl.when(j1 < trips)
            def _():
                idx_phase(1)

                @pl.when(j1 + 1 < trips)
                def _():
                    fire_in(0, j1 + 1)

            msg_phase(0)

        @pl.when(lax.rem(trips, 2) == 0)
        def _():
            msg_phase(1)

        plsc.subcore_barrier()
        _write_out(acc, out_hbm, cid, sid)

    if warp:
        return k(pseudo_t, src, dst, t_pad, y_flat)
    return k(pseudo_t, src, dst, y_flat)


def _elu(x):
    return jnp.where(x > 0, x, jnp.exp(x) - 1.0)


def _tc_h1(agg1, w1pad, root1, b1):
    def body(a_ref, w_ref, r_ref, b_ref, h_ref, inv_ref):
        asum = a_ref[0] + a_ref[1]
        deg = asum[:, 27:28]
        inv = 1.0 / jnp.maximum(deg, 1.0)
        inv_ref[...] = inv
        h_ref[...] = _elu(jnp.dot(asum, w_ref[...],
                                  preferred_element_type=jnp.float32) * inv
                          + r_ref[...] + b_ref[...])

    return pl.pallas_call(
        body,
        grid=(N // NB,),
        in_specs=[
            pl.BlockSpec((NC, NB, 32), lambda n: (0, n, 0)),
            pl.BlockSpec((32, 64), lambda n: (0, 0)),
            pl.BlockSpec((1, 64), lambda n: (0, 0)),
            pl.BlockSpec((64,), lambda n: (0,)),
        ],
        out_specs=[
            pl.BlockSpec((NB, 64), lambda n: (n, 0)),
            pl.BlockSpec((NB, 1), lambda n: (n, 0)),
        ],
        out_shape=[
            jax.ShapeDtypeStruct((N, 64), jnp.float32),
            jax.ShapeDtypeStruct((N, 1), jnp.float32),
        ],
    )(agg1, w1pad, root1, b1)


def _tc_table(x, w, out_dtype):
    kt, fin, fo = w.shape

    nb_per = N // NB

    def body(x_ref, w_ref, o_ref):
        o_ref[...] = jnp.dot(x_ref[...], w_ref[0],
                             preferred_element_type=jnp.float32
                             ).astype(out_dtype)

    return pl.pallas_call(
        body,
        grid=(nb_per, kt),
        in_specs=[
            pl.BlockSpec((NB, fin), lambda n, k_: (n, 0)),
            pl.BlockSpec((1, fin, fo), lambda n, k_: (k_, 0, 0)),
        ],
        out_specs=pl.BlockSpec((NB, fo), lambda n, k_: (k_ * nb_per + n, 0)),
        out_shape=jax.ShapeDtypeStruct((kt * N, fo), out_dtype),
    )(x, w)


def _tc_t(agg2, inv_deg, h1, root2, b2, w3, b3, w4p, b4p):
    def body(a_ref, i_ref, h_ref, r_ref, b2_ref, w3_ref, b3_ref,
             w4_ref, b4_ref, t_ref):
        h2 = _elu((a_ref[0] + a_ref[1]) * i_ref[...]
                  + jnp.dot(h_ref[...], r_ref[...],
                            preferred_element_type=jnp.float32)
                  + b2_ref[...])
        h3 = _elu(jnp.dot(h2, w3_ref[...],
                          preferred_element_type=jnp.float32) + b3_ref[...])
        t_ref[...] = jnp.dot(h3, w4_ref[...],
                             preferred_element_type=jnp.float32) + b4_ref[...]

    return pl.pallas_call(
        body,
        grid=(N // NB,),
        in_specs=[
            pl.BlockSpec((NC, NB, 64), lambda n: (0, n, 0)),
            pl.BlockSpec((NB, 1), lambda n: (n, 0)),
            pl.BlockSpec((NB, 64), lambda n: (n, 0)),
            pl.BlockSpec((64, 64), lambda n: (0, 0)),
            pl.BlockSpec((64,), lambda n: (0,)),
            pl.BlockSpec((64, 64), lambda n: (0, 0)),
            pl.BlockSpec((64,), lambda n: (0,)),
            pl.BlockSpec((64, 16), lambda n: (0, 0)),
            pl.BlockSpec((16,), lambda n: (0,)),
        ],
        out_specs=pl.BlockSpec((NB, 16), lambda n: (n, 0)),
        out_shape=jax.ShapeDtypeStruct((N, 16), jnp.float32),
    )(agg2, inv_deg, h1, root2, b2, w3, b3, w4p, b4p)


def _tc_final(agg3, inv_deg, x, root_w, bias):
    def body(a_ref, i_ref, x_ref, r_ref, b_ref, o_ref):
        o_ref[...] = ((a_ref[0] + a_ref[1]) * i_ref[...]
                      + jnp.dot(x_ref[...], r_ref[...],
                                preferred_element_type=jnp.float32)
                      + b_ref[...])

    return pl.pallas_call(
        body,
        grid=(N // NB,),
        in_specs=[
            pl.BlockSpec((NC, NB, 128), lambda n: (0, n, 0)),
            pl.BlockSpec((NB, 1), lambda n: (n, 0)),
            pl.BlockSpec((NB, 128), lambda n: (n, 0)),
            pl.BlockSpec((128, 128), lambda n: (0, 0)),
            pl.BlockSpec((128,), lambda n: (0,)),
        ],
        out_specs=pl.BlockSpec((NB, 128), lambda n: (n, 0)),
        out_shape=jax.ShapeDtypeStruct((N, 128), jnp.float32),
    )(agg3, inv_deg, x, root_w, bias)


def kernel(input, edge_index, pseudo, stn1_w, stn1_root, stn1_b,
           stn2_w, stn2_root, stn2_b, stn3_w, stn3_b, stn4_w, stn4_b,
           conv_w, conv_root, conv_b):
    src = edge_index[0]
    dst = edge_index[1]
    pseudo_t = pseudo.T

    y3 = _tc_table(input, conv_w, jnp.bfloat16)

    agg1 = _sc_stn1(pseudo_t, dst)
    w1pad = jnp.zeros((32, 64), jnp.float32).at[:KT].set(
        stn1_w.reshape(KT, 64))
    h1, inv_deg = _tc_h1(agg1, w1pad, stn1_root, stn1_b)

    y2 = _tc_table(h1, stn2_w, jnp.float32)
    agg2 = _sc_gather_conv(pseudo_t, src, dst, y2, 64, 32)

    w4p = jnp.zeros((64, 16), jnp.float32).at[:, :DIM].set(stn4_w)
    b4p = jnp.zeros((16,), jnp.float32).at[:DIM].set(stn4_b)
    t_pad = _tc_t(agg2, inv_deg, h1, stn2_root, stn2_b,
                  stn3_w, stn3_b, w4p, b4p)

    agg3 = _sc_gather_conv(pseudo_t, src, dst, y3, 128, 32, t_pad=t_pad)

    return _tc_final(agg3, inv_deg, input, conv_root, conv_b)


# speedup vs baseline: 1.0961x; 1.0502x over previous
import functools

import jax
import jax.numpy as jnp
from jax import lax
from jax.experimental import pallas as pl
from jax.experimental.pallas import tpu as pltpu
from jax.experimental.pallas import tpu_sc as plsc

N = 10000
E = 160000
DIM = 3
K = 3
S = 2 ** DIM
KT = K ** DIM
C = 32
NCHUNK = E // C
NC = 2
NS = 16
NW = NC * NS
L = 16
ROWS_PER_TILE = N // NS
NB = 1000


def _basis_from_v(vs):
    frs, los = [], []
    for v in vs:
        ge1 = v >= 1.0
        lof = jnp.where(ge1, jnp.float32(1.0), jnp.float32(0.0))
        frs.append(v - lof)
        los.append(jnp.where(ge1, jnp.int32(1), jnp.int32(0)))
    bs_list, ws_list = [], []
    for s_ in range(S):
        bits = [(s_ >> d) & 1 for d in range(DIM)]
        b = None
        w = None
        for d in range(DIM):
            f = frs[d] if bits[d] else (1.0 - frs[d])
            b = f if b is None else b * f
            term = (los[d] + bits[d]) * (K ** d)
            w = term if w is None else w + term
        bs_list.append(b)
        ws_list.append(w)
    return bs_list, ws_list


ZROWS = 8
NUNITS = N // ZROWS


def _zero_acc(acc, zbuf, sid, width):
    zv = jnp.zeros((L,), jnp.float32)
    for j in range(ZROWS):
        for q in range(width // L):
            zbuf[j, pl.ds(q * L, L)] = zv

    @pl.loop(sid, NUNITS, step=NS)
    def _(u):
        pltpu.sync_copy(zbuf, acc.at[pl.ds(u * ZROWS, ZROWS)])


def _write_out(acc, out_hbm, cid, sid):
    @pl.loop(sid, NUNITS, step=NS)
    def _(u):
        sl = pl.ds(u * ZROWS, ZROWS)
        pltpu.sync_copy(acc.at[sl], out_hbm.at[cid, sl])


CA = 128


def _sc_stn1(pseudo_t, dst):
    mesh = plsc.VectorSubcoreMesh(core_axis_name="c", subcore_axis_name="s")
    nch = E // CA

    @functools.partial(
        pl.kernel,
        out_type=jax.ShapeDtypeStruct((NC, N, 32), jnp.float32),
        mesh=mesh,
        compiler_params=pltpu.CompilerParams(needs_layout_passes=False, use_tc_tiling_on_sc=False),
        scratch_types=[
            pltpu.VMEM_SHARED((N, 32), jnp.float32),
            pltpu.VMEM((DIM, CA), jnp.float32),
            pltpu.VMEM((CA,), jnp.int32),
            pltpu.VMEM((CA, 32), jnp.float32),
            pltpu.VMEM((ZROWS, 32), jnp.float32),
        ],
    )
    def k(p_hbm, dst_hbm, out_hbm, acc, p_v, dst_v, msg_v, zbuf):
        cid = lax.axis_index("c")
        sid = lax.axis_index("s")
        wid = sid * NC + cid

        _zero_acc(acc, zbuf, sid, 32)
        plsc.subcore_barrier()

        iota = lax.iota(jnp.int32, L)
        zv = jnp.zeros((L,), jnp.float32)
        onecol = jnp.where(iota == 11, jnp.float32(1.0), jnp.float32(0.0))

        @pl.loop(wid, nch, step=NW)
        def _(cidx):
            ebase = cidx * CA
            pltpu.sync_copy(p_hbm.at[:, pl.ds(ebase, CA)], p_v)
            pltpu.sync_copy(dst_hbm.at[pl.ds(ebase, CA)], dst_v)
            for e in range(CA):
                msg_v[e, pl.ds(0, L)] = zv
                msg_v[e, pl.ds(L, L)] = onecol
            for g in range(CA // L):
                gs = pl.ds(g * L, L)
                vs = [p_v[d, gs] * jnp.float32(K - 1) for d in range(DIM)]
                bs_list, ws_list = _basis_from_v(vs)
                rows16 = iota + g * L
                for s_ in range(S):
                    plsc.store_scatter(msg_v, [rows16, ws_list[s_]],
                                       bs_list[s_])
            pltpu.sync_copy(msg_v, acc.at[dst_v], add=True)

        plsc.subcore_barrier()
        _write_out(acc, out_hbm, cid, sid)

    return k(pseudo_t, dst)


def _sc_gather_conv(pseudo_t, src, dst, y_flat, fout, c_sz, t_pad=None):
    warp = t_pad is not None
    bf16 = y_flat.dtype == jnp.bfloat16
    nq = fout // L
    nch = E // c_sz
    mesh = plsc.VectorSubcoreMesh(core_axis_name="c", subcore_axis_name="s")

    slot_scratch = [
        pltpu.VMEM((DIM, c_sz), jnp.float32),
        pltpu.VMEM((c_sz,), jnp.int32),
        pltpu.VMEM((c_sz,), jnp.int32),
        pltpu.VMEM((S * c_sz,), jnp.float32),
        pltpu.VMEM((S * c_sz,), jnp.int32),
        pltpu.VMEM((S * c_sz, fout),
                   jnp.bfloat16 if bf16 else jnp.float32),
        pltpu.SemaphoreType.DMA,
        pltpu.SemaphoreType.DMA,
        pltpu.VMEM((c_sz,), jnp.int32),
    ]
    if warp:
        slot_scratch += [
            pltpu.VMEM((c_sz, 16), jnp.float32),
            pltpu.VMEM((c_sz, 16), jnp.float32),
            pltpu.SemaphoreType.DMA,
        ]
    nslot = len(slot_scratch)

    @functools.partial(
        pl.kernel,
        out_type=jax.ShapeDtypeStruct((NC, N, fout), jnp.float32),
        mesh=mesh,
        compiler_params=pltpu.CompilerParams(
            needs_layout_passes=False, use_tc_tiling_on_sc=False),
        scratch_types=[
            pltpu.VMEM_SHARED((N, fout), jnp.float32),
            pltpu.VMEM((c_sz, fout), jnp.float32),
            pltpu.VMEM((ZROWS, fout), jnp.float32),
        ] + slot_scratch * 2,
    )
    def k(*refs):
        if warp:
            p_hbm, src_hbm, dst_hbm, t_hbm, y_hbm, out_hbm = refs[:6]
            rest = refs[6:]
        else:
            p_hbm, src_hbm, dst_hbm, y_hbm, out_hbm = refs[:5]
            rest = refs[5:]
        acc, msg_v, zbuf = rest[:3]
        slots = [rest[3:3 + nslot], rest[3 + nslot:3 + 2 * nslot]]

        cid = lax.axis_index("c")
        sid = lax.axis_index("s")
        wid = sid * NC + cid
        trips = (nch - 1 - wid) // NW + 1

        _zero_acc(acc, zbuf, sid, fout)
        plsc.subcore_barrier()

        iota = lax.iota(jnp.int32, L)

        def fire_in(b, j):
            p_v, src_v, dst_v = slots[b][:3]
            sem_in = slots[b][6]
            ebase = (wid + j * NW) * c_sz
            pltpu.async_copy(p_hbm.at[:, pl.ds(ebase, c_sz)], p_v, sem_in)
            pltpu.async_copy(src_hbm.at[pl.ds(ebase, c_sz)], src_v, sem_in)
            pltpu.async_copy(dst_hbm.at[pl.ds(ebase, c_sz)], dst_v, sem_in)

        def idx_phase(b):
            p_v, src_v, dst_v, bbuf, idx_v, rows_v, sem_in, sem_g = \
                slots[b][:8]
            pltpu.make_async_copy(p_hbm.at[:, pl.ds(0, c_sz)], p_v,
                                  sem_in).wait()
            pltpu.make_async_copy(src_hbm.at[pl.ds(0, c_sz)], src_v,
                                  sem_in).wait()
            pltpu.make_async_copy(dst_hbm.at[pl.ds(0, c_sz)], dst_v,
                                  sem_in).wait()
            if warp:
                tsrc_v, tdst_v, sem_t = slots[b][9:12]
                pltpu.async_copy(t_hbm.at[src_v], tsrc_v, sem_t)
                pltpu.async_copy(t_hbm.at[dst_v], tdst_v, sem_t)
                pltpu.make_async_copy(t_hbm.at[src_v], tsrc_v, sem_t).wait()
                pltpu.make_async_copy(t_hbm.at[dst_v], tdst_v, sem_t).wait()
            dst2_v = slots[b][8]
            for g in range(c_sz // L):
                gs = pl.ds(g * L, L)
                sv = src_v[gs]
                dst2_v[gs] = dst_v[gs]
                if warp:
                    rows16 = iota + g * L
                    vs = []
                    for d in range(DIM):
                        dvec = jnp.full((L,), d, jnp.int32)
                        ts = plsc.load_gather(tsrc_v, [rows16, dvec])
                        td = plsc.load_gather(tdst_v, [rows16, dvec])
                        npd = jnp.clip(p_v[d, gs] + td - ts,
                                       jnp.float32(0.0), jnp.float32(1.0))
                        vs.append(npd * jnp.float32(K - 1))
                else:
                    vs = [p_v[d, gs] * jnp.float32(K - 1)
                          for d in range(DIM)]
                bs_list, ws_list = _basis_from_v(vs)
                for s_ in range(S):
                    bbuf[pl.ds(s_ * c_sz + g * L, L)] = bs_list[s_]
                    idx_v[pl.ds(s_ * c_sz + g * L, L)] = \
                        ws_list[s_] * N + sv
            pltpu.async_copy(y_hbm.at[idx_v], rows_v, sem_g)

        def msg_phase(b):
            bbuf, idx_v, rows_v = slots[b][3:6]
            sem_g = slots[b][7]
            dst2_v = slots[b][8]
            pltpu.make_async_copy(y_hbm.at[idx_v], rows_v, sem_g).wait()

            nh = fout // 32

            @pl.loop(0, c_sz // 4)
            def _(e4):
                for u in range(4):
                    e = e4 * 4 + u
                    evec = jnp.full((L,), e, jnp.int32)
                    if bf16:
                        acce = [jnp.zeros((L,), jnp.float32)
                                for _ in range(nh)]
                        acco = [jnp.zeros((L,), jnp.float32)
                                for _ in range(nh)]
                        for s_ in range(S):
                            b_ = plsc.load_gather(bbuf,
                                                  [evec + s_ * c_sz])
                            for h in range(nh):
                                row32 = rows_v[s_ * c_sz + e,
                                               pl.ds(h * 32, 32)]
                                ev, od = plsc.unpack(
                                    row32,
                                    format=plsc.PackFormat.INTERLEAVED)
                                acce[h] = acce[h] + b_ * ev
                                acco[h] = acco[h] + b_ * od
                        for h in range(nh):
                            cbase = 32 * h + 2 * iota
                            plsc.store_scatter(msg_v, [evec, cbase],
                                               acce[h])
                            plsc.store_scatter(msg_v, [evec, cbase + 1],
                                               acco[h])
                    else:
                        accs = [jnp.zeros((L,), jnp.float32)
                                for _ in range(nq)]
                        for s_ in range(S):
                            b_ = plsc.load_gather(bbuf,
                                                  [evec + s_ * c_sz])
                            for q in range(nq):
                                row = rows_v[s_ * c_sz + e,
                                             pl.ds(q * L, L)]
                                accs[q] = accs[q] + b_ * row
                        for q in range(nq):
                            msg_v[e, pl.ds(q * L, L)] = accs[q]

            pltpu.sync_copy(msg_v, acc.at[dst2_v], add=True)

        fire_in(0, 0)
        npairs = (trips + 1) // 2

        @pl.loop(0, npairs)
        def _(pp):
            j1 = 2 * pp + 1
            idx_phase(0)

            @pl.when(j1 < trips)
            def _():
                fire_in(1, j1)

            @pl.when(pp > 0)
            def _():
                msg_phase(1)

            @pl.when(j1 < trips)
            def _():
                idx_phase(1)

                @pl.when(j1 + 1 < trips)
                def _():
                    fire_in(0, j1 + 1)

            msg_phase(0)

        @pl.when(lax.rem(trips, 2) == 0)
        def _():
            msg_phase(1)

        plsc.subcore_barrier()
        _write_out(acc, out_hbm, cid, sid)

    if warp:
        return k(pseudo_t, src, dst, t_pad, y_flat)
    return k(pseudo_t, src, dst, y_flat)


def _elu(x):
    return jnp.where(x > 0, x, jnp.exp(x) - 1.0)


def _tc_h1(agg1, w1pad, root1, b1):
    def body(a_ref, w_ref, r_ref, b_ref, h_ref, inv_ref):
        asum = a_ref[0] + a_ref[1]
        deg = asum[:, 27:28]
        inv = 1.0 / jnp.maximum(deg, 1.0)
        inv_ref[...] = inv
        h_ref[...] = _elu(jnp.dot(asum, w_ref[...],
                                  preferred_element_type=jnp.float32) * inv
                          + r_ref[...] + b_ref[...])

    return pl.pallas_call(
        body,
        grid=(N // NB,),
        in_specs=[
            pl.BlockSpec((NC, NB, 32), lambda n: (0, n, 0)),
            pl.BlockSpec((32, 64), lambda n: (0, 0)),
            pl.BlockSpec((1, 64), lambda n: (0, 0)),
            pl.BlockSpec((64,), lambda n: (0,)),
        ],
        out_specs=[
            pl.BlockSpec((NB, 64), lambda n: (n, 0)),
            pl.BlockSpec((NB, 1), lambda n: (n, 0)),
        ],
        out_shape=[
            jax.ShapeDtypeStruct((N, 64), jnp.float32),
            jax.ShapeDtypeStruct((N, 1), jnp.float32),
        ],
    )(agg1, w1pad, root1, b1)


def _tc_table(x, w, out_dtype):
    kt, fin, fo = w.shape

    nb_per = N // NB

    def body(x_ref, w_ref, o_ref):
        o_ref[...] = jnp.dot(x_ref[...], w_ref[0],
                             preferred_element_type=jnp.float32
                             ).astype(out_dtype)

    return pl.pallas_call(
        body,
        grid=(nb_per, kt),
        in_specs=[
            pl.BlockSpec((NB, fin), lambda n, k_: (n, 0)),
            pl.BlockSpec((1, fin, fo), lambda n, k_: (k_, 0, 0)),
        ],
        out_specs=pl.BlockSpec((NB, fo), lambda n, k_: (k_ * nb_per + n, 0)),
        out_shape=jax.ShapeDtypeStruct((kt * N, fo), out_dtype),
    )(x, w)


def _tc_t(agg2, inv_deg, h1, root2, b2, w3, b3, w4p, b4p):
    def body(a_ref, i_ref, h_ref, r_ref, b2_ref, w3_ref, b3_ref,
             w4_ref, b4_ref, t_ref):
        h2 = _elu((a_ref[0] + a_ref[1]) * i_ref[...]
                  + jnp.dot(h_ref[...], r_ref[...],
                            preferred_element_type=jnp.float32)
                  + b2_ref[...])
        h3 = _elu(jnp.dot(h2, w3_ref[...],
                          preferred_element_type=jnp.float32) + b3_ref[...])
        t_ref[...] = jnp.dot(h3, w4_ref[...],
                             preferred_element_type=jnp.float32) + b4_ref[...]

    return pl.pallas_call(
        body,
        grid=(N // NB,),
        in_specs=[
            pl.BlockSpec((NC, NB, 64), lambda n: (0, n, 0)),
            pl.BlockSpec((NB, 1), lambda n: (n, 0)),
            pl.BlockSpec((NB, 64), lambda n: (n, 0)),
            pl.BlockSpec((64, 64), lambda n: (0, 0)),
            pl.BlockSpec((64,), lambda n: (0,)),
            pl.BlockSpec((64, 64), lambda n: (0, 0)),
            pl.BlockSpec((64,), lambda n: (0,)),
            pl.BlockSpec((64, 16), lambda n: (0, 0)),
            pl.BlockSpec((16,), lambda n: (0,)),
        ],
        out_specs=pl.BlockSpec((NB, 16), lambda n: (n, 0)),
        out_shape=jax.ShapeDtypeStruct((N, 16), jnp.float32),
    )(agg2, inv_deg, h1, root2, b2, w3, b3, w4p, b4p)


def _tc_final(agg3, inv_deg, x, root_w, bias):
    def body(a_ref, i_ref, x_ref, r_ref, b_ref, o_ref):
        o_ref[...] = ((a_ref[0] + a_ref[1]) * i_ref[...]
                      + jnp.dot(x_ref[...], r_ref[...],
                                preferred_element_type=jnp.float32)
                      + b_ref[...])

    return pl.pallas_call(
        body,
        grid=(N // NB,),
        in_specs=[
            pl.BlockSpec((NC, NB, 128), lambda n: (0, n, 0)),
            pl.BlockSpec((NB, 1), lambda n: (n, 0)),
            pl.BlockSpec((NB, 128), lambda n: (n, 0)),
            pl.BlockSpec((128, 128), lambda n: (0, 0)),
            pl.BlockSpec((128,), lambda n: (0,)),
        ],
        out_specs=pl.BlockSpec((NB, 128), lambda n: (n, 0)),
        out_shape=jax.ShapeDtypeStruct((N, 128), jnp.float32),
    )(agg3, inv_deg, x, root_w, bias)


def kernel(input, edge_index, pseudo, stn1_w, stn1_root, stn1_b,
           stn2_w, stn2_root, stn2_b, stn3_w, stn3_b, stn4_w, stn4_b,
           conv_w, conv_root, conv_b):
    src = edge_index[0]
    dst = edge_index[1]
    pseudo_t = pseudo.T

    y3 = _tc_table(input, conv_w, jnp.float32)

    agg1 = _sc_stn1(pseudo_t, dst)
    w1pad = jnp.zeros((32, 64), jnp.float32).at[:KT].set(
        stn1_w.reshape(KT, 64))
    h1, inv_deg = _tc_h1(agg1, w1pad, stn1_root, stn1_b)

    y2 = _tc_table(h1, stn2_w, jnp.float32)
    agg2 = _sc_gather_conv(pseudo_t, src, dst, y2, 64, 32)

    w4p = jnp.zeros((64, 16), jnp.float32).at[:, :DIM].set(stn4_w)
    b4p = jnp.zeros((16,), jnp.float32).at[:DIM].set(stn4_b)
    t_pad = _tc_t(agg2, inv_deg, h1, stn2_root, stn2_b,
                  stn3_w, stn3_b, w4p, b4p)

    agg3 = _sc_gather_conv(pseudo_t, src, dst, y3, 128, 16, t_pad=t_pad)

    return _tc_final(agg3, inv_deg, input, conv_root, conv_b)


# speedup vs baseline: 1.1995x; 1.0943x over previous
import functools

import jax
import jax.numpy as jnp
from jax import lax
from jax.experimental import pallas as pl
from jax.experimental.pallas import tpu as pltpu
from jax.experimental.pallas import tpu_sc as plsc

N = 10000
E = 160000
DIM = 3
K = 3
S = 2 ** DIM
KT = K ** DIM
C = 32
NCHUNK = E // C
NC = 2
NS = 16
NW = NC * NS
L = 16
ROWS_PER_TILE = N // NS
NB = 1000


def _basis_from_v(vs):
    frs, los = [], []
    for v in vs:
        ge1 = v >= 1.0
        lof = jnp.where(ge1, jnp.float32(1.0), jnp.float32(0.0))
        frs.append(v - lof)
        los.append(jnp.where(ge1, jnp.int32(1), jnp.int32(0)))
    bs_list, ws_list = [], []
    for s_ in range(S):
        bits = [(s_ >> d) & 1 for d in range(DIM)]
        b = None
        w = None
        for d in range(DIM):
            f = frs[d] if bits[d] else (1.0 - frs[d])
            b = f if b is None else b * f
            term = (los[d] + bits[d]) * (K ** d)
            w = term if w is None else w + term
        bs_list.append(b)
        ws_list.append(w)
    return bs_list, ws_list


ZROWS = 8
NUNITS = N // ZROWS


def _zero_acc(acc, zbuf, sid, width):
    zv = jnp.zeros((L,), jnp.float32)
    for j in range(ZROWS):
        for q in range(width // L):
            zbuf[j, pl.ds(q * L, L)] = zv

    @pl.loop(sid, NUNITS, step=NS)
    def _(u):
        pltpu.sync_copy(zbuf, acc.at[pl.ds(u * ZROWS, ZROWS)])


def _write_out(acc, out_hbm, cid, sid):
    @pl.loop(sid, NUNITS, step=NS)
    def _(u):
        sl = pl.ds(u * ZROWS, ZROWS)
        pltpu.sync_copy(acc.at[sl], out_hbm.at[cid, sl])


CA = 128


def _sc_stn1(pseudo_t, dst):
    mesh = plsc.VectorSubcoreMesh(core_axis_name="c", subcore_axis_name="s")
    nch = E // CA

    @functools.partial(
        pl.kernel,
        out_type=jax.ShapeDtypeStruct((NC, N, 32), jnp.float32),
        mesh=mesh,
        compiler_params=pltpu.CompilerParams(needs_layout_passes=False, use_tc_tiling_on_sc=False),
        scratch_types=[
            pltpu.VMEM_SHARED((N, 32), jnp.float32),
            pltpu.VMEM((DIM, CA), jnp.float32),
            pltpu.VMEM((CA,), jnp.int32),
            pltpu.VMEM((CA, 32), jnp.float32),
            pltpu.VMEM((ZROWS, 32), jnp.float32),
        ],
    )
    def k(p_hbm, dst_hbm, out_hbm, acc, p_v, dst_v, msg_v, zbuf):
        cid = lax.axis_index("c")
        sid = lax.axis_index("s")
        wid = sid * NC + cid

        _zero_acc(acc, zbuf, sid, 32)
        plsc.subcore_barrier()

        iota = lax.iota(jnp.int32, L)
        zv = jnp.zeros((L,), jnp.float32)
        onecol = jnp.where(iota == 11, jnp.float32(1.0), jnp.float32(0.0))

        @pl.loop(wid, nch, step=NW)
        def _(cidx):
            ebase = cidx * CA
            pltpu.sync_copy(p_hbm.at[:, pl.ds(ebase, CA)], p_v)
            pltpu.sync_copy(dst_hbm.at[pl.ds(ebase, CA)], dst_v)
            for e in range(CA):
                msg_v[e, pl.ds(0, L)] = zv
                msg_v[e, pl.ds(L, L)] = onecol
            for g in range(CA // L):
                gs = pl.ds(g * L, L)
                vs = [p_v[d, gs] * jnp.float32(K - 1) for d in range(DIM)]
                bs_list, ws_list = _basis_from_v(vs)
                rows16 = iota + g * L
                for s_ in range(S):
                    plsc.store_scatter(msg_v, [rows16, ws_list[s_]],
                                       bs_list[s_])
            pltpu.sync_copy(msg_v, acc.at[dst_v], add=True)

        plsc.subcore_barrier()
        _write_out(acc, out_hbm, cid, sid)

    return k(pseudo_t, dst)


def _sc_gather_conv(pseudo_t, src, dst, y_flat, fout, c_sz, t_pad=None):
    warp = t_pad is not None
    bf16 = y_flat.dtype == jnp.bfloat16
    nq = fout // L
    nch = E // c_sz
    mesh = plsc.VectorSubcoreMesh(core_axis_name="c", subcore_axis_name="s")

    slot_scratch = [
        pltpu.VMEM((DIM, c_sz), jnp.float32),
        pltpu.VMEM((c_sz,), jnp.int32),
        pltpu.VMEM((c_sz,), jnp.int32),
        pltpu.VMEM((S * c_sz,), jnp.float32),
        pltpu.VMEM((S * c_sz,), jnp.int32),
        pltpu.VMEM((S * c_sz, fout),
                   jnp.bfloat16 if bf16 else jnp.float32),
        pltpu.SemaphoreType.DMA,
        pltpu.SemaphoreType.DMA,
        pltpu.VMEM((c_sz,), jnp.int32),
    ]
    if warp:
        slot_scratch += [
            pltpu.VMEM((c_sz, 16), jnp.float32),
            pltpu.VMEM((c_sz, 16), jnp.float32),
            pltpu.SemaphoreType.DMA,
        ]
    nslot = len(slot_scratch)

    @functools.partial(
        pl.kernel,
        out_type=jax.ShapeDtypeStruct((NC, N, fout), jnp.float32),
        mesh=mesh,
        compiler_params=pltpu.CompilerParams(
            needs_layout_passes=False, use_tc_tiling_on_sc=False),
        scratch_types=[
            pltpu.VMEM_SHARED((N, fout), jnp.float32),
            pltpu.VMEM((c_sz, fout), jnp.float32),
            pltpu.VMEM((ZROWS, fout), jnp.float32),
        ] + slot_scratch * 2,
    )
    def k(*refs):
        if warp:
            p_hbm, src_hbm, dst_hbm, t_hbm, y_hbm, out_hbm = refs[:6]
            rest = refs[6:]
        else:
            p_hbm, src_hbm, dst_hbm, y_hbm, out_hbm = refs[:5]
            rest = refs[5:]
        acc, msg_v, zbuf = rest[:3]
        slots = [rest[3:3 + nslot], rest[3 + nslot:3 + 2 * nslot]]

        cid = lax.axis_index("c")
        sid = lax.axis_index("s")
        wid = sid * NC + cid
        trips = (nch - 1 - wid) // NW + 1

        _zero_acc(acc, zbuf, sid, fout)
        plsc.subcore_barrier()

        iota = lax.iota(jnp.int32, L)

        def fire_in(b, j):
            p_v, src_v, dst_v = slots[b][:3]
            sem_in = slots[b][6]
            ebase = (wid + j * NW) * c_sz
            pltpu.async_copy(p_hbm.at[:, pl.ds(ebase, c_sz)], p_v, sem_in)
            pltpu.async_copy(src_hbm.at[pl.ds(ebase, c_sz)], src_v, sem_in)
            pltpu.async_copy(dst_hbm.at[pl.ds(ebase, c_sz)], dst_v, sem_in)

        def idx_phase(b):
            p_v, src_v, dst_v, bbuf, idx_v, rows_v, sem_in, sem_g = \
                slots[b][:8]
            pltpu.make_async_copy(p_hbm.at[:, pl.ds(0, c_sz)], p_v,
                                  sem_in).wait()
            pltpu.make_async_copy(src_hbm.at[pl.ds(0, c_sz)], src_v,
                                  sem_in).wait()
            pltpu.make_async_copy(dst_hbm.at[pl.ds(0, c_sz)], dst_v,
                                  sem_in).wait()
            if warp:
                tsrc_v, tdst_v, sem_t = slots[b][9:12]
                pltpu.async_copy(t_hbm.at[src_v], tsrc_v, sem_t)
                pltpu.async_copy(t_hbm.at[dst_v], tdst_v, sem_t)
                pltpu.make_async_copy(t_hbm.at[src_v], tsrc_v, sem_t).wait()
                pltpu.make_async_copy(t_hbm.at[dst_v], tdst_v, sem_t).wait()
            dst2_v = slots[b][8]
            for g in range(c_sz // L):
                gs = pl.ds(g * L, L)
                sv = src_v[gs]
                dst2_v[gs] = dst_v[gs]
                if warp:
                    rows16 = iota + g * L
                    vs = []
                    for d in range(DIM):
                        dvec = jnp.full((L,), d, jnp.int32)
                        ts = plsc.load_gather(tsrc_v, [rows16, dvec])
                        td = plsc.load_gather(tdst_v, [rows16, dvec])
                        npd = jnp.clip(p_v[d, gs] + td - ts,
                                       jnp.float32(0.0), jnp.float32(1.0))
                        vs.append(npd * jnp.float32(K - 1))
                else:
                    vs = [p_v[d, gs] * jnp.float32(K - 1)
                          for d in range(DIM)]
                bs_list, ws_list = _basis_from_v(vs)
                for s_ in range(S):
                    bbuf[pl.ds(s_ * c_sz + g * L, L)] = bs_list[s_]
                    idx_v[pl.ds(s_ * c_sz + g * L, L)] = \
                        ws_list[s_] * N + sv
            pltpu.async_copy(y_hbm.at[idx_v], rows_v, sem_g)

        def msg_phase(b):
            bbuf, idx_v, rows_v = slots[b][3:6]
            sem_g = slots[b][7]
            dst2_v = slots[b][8]
            pltpu.make_async_copy(y_hbm.at[idx_v], rows_v, sem_g).wait()

            nh = fout // 32

            @pl.loop(0, c_sz // 4)
            def _(e4):
                for u in range(4):
                    e = e4 * 4 + u
                    evec = jnp.full((L,), e, jnp.int32)
                    if bf16:
                        acce = [jnp.zeros((L,), jnp.float32)
                                for _ in range(nh)]
                        acco = [jnp.zeros((L,), jnp.float32)
                                for _ in range(nh)]
                        for s_ in range(S):
                            b_ = plsc.load_gather(bbuf,
                                                  [evec + s_ * c_sz])
                            for h in range(nh):
                                row32 = rows_v[s_ * c_sz + e,
                                               pl.ds(h * 32, 32)]
                                ev, od = plsc.unpack(
                                    row32,
                                    format=plsc.PackFormat.INTERLEAVED)
                                acce[h] = acce[h] + b_ * ev
                                acco[h] = acco[h] + b_ * od
                        for h in range(nh):
                            cbase = 32 * h + 2 * iota
                            plsc.store_scatter(msg_v, [evec, cbase],
                                               acce[h])
                            plsc.store_scatter(msg_v, [evec, cbase + 1],
                                               acco[h])
                    else:
                        accs = [jnp.zeros((L,), jnp.float32)
                                for _ in range(nq)]
                        for s_ in range(S):
                            b_ = plsc.load_gather(bbuf,
                                                  [evec + s_ * c_sz])
                            for q in range(nq):
                                row = rows_v[s_ * c_sz + e,
                                             pl.ds(q * L, L)]
                                accs[q] = accs[q] + b_ * row
                        for q in range(nq):
                            msg_v[e, pl.ds(q * L, L)] = accs[q]

            pltpu.sync_copy(msg_v, acc.at[dst2_v], add=True)

        fire_in(0, 0)
        npairs = (trips + 1) // 2

        @pl.loop(0, npairs)
        def _(pp):
            j1 = 2 * pp + 1
            idx_phase(0)

            @pl.when(j1 < trips)
            def _():
                fire_in(1, j1)

            @pl.when(pp > 0)
            def _():
                msg_phase(1)

            @pl.when(j1 < trips)
            def _():
                idx_phase(1)

                @pl.when(j1 + 1 < trips)
                def _():
                    fire_in(0, j1 + 1)

            msg_phase(0)

        @pl.when(lax.rem(trips, 2) == 0)
        def _():
            msg_phase(1)

        plsc.subcore_barrier()
        _write_out(acc, out_hbm, cid, sid)

    if warp:
        return k(pseudo_t, src, dst, t_pad, y_flat)
    return k(pseudo_t, src, dst, y_flat)


def _elu(x):
    return jnp.where(x > 0, x, jnp.exp(x) - 1.0)


def _tc_h1(agg1, w1pad, root1, b1):
    def body(a_ref, w_ref, r_ref, b_ref, h_ref, inv_ref):
        asum = a_ref[0] + a_ref[1]
        deg = asum[:, 27:28]
        inv = 1.0 / jnp.maximum(deg, 1.0)
        inv_ref[...] = inv
        h_ref[...] = _elu(jnp.dot(asum, w_ref[...],
                                  preferred_element_type=jnp.float32) * inv
                          + r_ref[...] + b_ref[...])

    return pl.pallas_call(
        body,
        grid=(N // NB,),
        in_specs=[
            pl.BlockSpec((NC, NB, 32), lambda n: (0, n, 0)),
            pl.BlockSpec((32, 64), lambda n: (0, 0)),
            pl.BlockSpec((1, 64), lambda n: (0, 0)),
            pl.BlockSpec((64,), lambda n: (0,)),
        ],
        out_specs=[
            pl.BlockSpec((NB, 64), lambda n: (n, 0)),
            pl.BlockSpec((NB, 1), lambda n: (n, 0)),
        ],
        out_shape=[
            jax.ShapeDtypeStruct((N, 64), jnp.float32),
            jax.ShapeDtypeStruct((N, 1), jnp.float32),
        ],
    )(agg1, w1pad, root1, b1)


def _tc_table(x, w, out_dtype):
    kt, fin, fo = w.shape
    kb = 9

    def body(x_ref, w_ref, o_ref):
        xb = x_ref[...]
        for ki in range(kb):
            o_ref[ki] = jnp.dot(xb, w_ref[ki],
                                preferred_element_type=jnp.float32
                                ).astype(out_dtype)

    out = pl.pallas_call(
        body,
        grid=(N // NB, kt // kb),
        in_specs=[
            pl.BlockSpec((NB, fin), lambda n, k_: (n, 0)),
            pl.BlockSpec((kb, fin, fo), lambda n, k_: (k_, 0, 0)),
        ],
        out_specs=pl.BlockSpec((kb, NB, fo), lambda n, k_: (k_, n, 0)),
        out_shape=jax.ShapeDtypeStruct((kt, N, fo), out_dtype),
    )(x, w)
    return out.reshape(kt * N, fo)


def _tc_t(agg2, inv_deg, h1, root2, b2, w3, b3, w4p, b4p):
    def body(a_ref, i_ref, h_ref, r_ref, b2_ref, w3_ref, b3_ref,
             w4_ref, b4_ref, t_ref):
        h2 = _elu((a_ref[0] + a_ref[1]) * i_ref[...]
                  + jnp.dot(h_ref[...], r_ref[...],
                            preferred_element_type=jnp.float32)
                  + b2_ref[...])
        h3 = _elu(jnp.dot(h2, w3_ref[...],
                          preferred_element_type=jnp.float32) + b3_ref[...])
        t_ref[...] = jnp.dot(h3, w4_ref[...],
                             preferred_element_type=jnp.float32) + b4_ref[...]

    return pl.pallas_call(
        body,
        grid=(N // NB,),
        in_specs=[
            pl.BlockSpec((NC, NB, 64), lambda n: (0, n, 0)),
            pl.BlockSpec((NB, 1), lambda n: (n, 0)),
            pl.BlockSpec((NB, 64), lambda n: (n, 0)),
            pl.BlockSpec((64, 64), lambda n: (0, 0)),
            pl.BlockSpec((64,), lambda n: (0,)),
            pl.BlockSpec((64, 64), lambda n: (0, 0)),
            pl.BlockSpec((64,), lambda n: (0,)),
            pl.BlockSpec((64, 16), lambda n: (0, 0)),
            pl.BlockSpec((16,), lambda n: (0,)),
        ],
        out_specs=pl.BlockSpec((NB, 16), lambda n: (n, 0)),
        out_shape=jax.ShapeDtypeStruct((N, 16), jnp.float32),
    )(agg2, inv_deg, h1, root2, b2, w3, b3, w4p, b4p)


def _tc_final(agg3, inv_deg, x, root_w, bias):
    def body(a_ref, i_ref, x_ref, r_ref, b_ref, o_ref):
        o_ref[...] = ((a_ref[0] + a_ref[1]) * i_ref[...]
                      + jnp.dot(x_ref[...], r_ref[...],
                                preferred_element_type=jnp.float32)
                      + b_ref[...])

    return pl.pallas_call(
        body,
        grid=(N // NB,),
        in_specs=[
            pl.BlockSpec((NC, NB, 128), lambda n: (0, n, 0)),
            pl.BlockSpec((NB, 1), lambda n: (n, 0)),
            pl.BlockSpec((NB, 128), lambda n: (n, 0)),
            pl.BlockSpec((128, 128), lambda n: (0, 0)),
            pl.BlockSpec((128,), lambda n: (0,)),
        ],
        out_specs=pl.BlockSpec((NB, 128), lambda n: (n, 0)),
        out_shape=jax.ShapeDtypeStruct((N, 128), jnp.float32),
    )(agg3, inv_deg, x, root_w, bias)


def kernel(input, edge_index, pseudo, stn1_w, stn1_root, stn1_b,
           stn2_w, stn2_root, stn2_b, stn3_w, stn3_b, stn4_w, stn4_b,
           conv_w, conv_root, conv_b):
    src = edge_index[0]
    dst = edge_index[1]
    pseudo_t = pseudo.T

    y3 = _tc_table(input, conv_w, jnp.float32)

    agg1 = _sc_stn1(pseudo_t, dst)
    w1pad = jnp.zeros((32, 64), jnp.float32).at[:KT].set(
        stn1_w.reshape(KT, 64))
    h1, inv_deg = _tc_h1(agg1, w1pad, stn1_root, stn1_b)

    y2 = _tc_table(h1, stn2_w, jnp.float32)
    agg2 = _sc_gather_conv(pseudo_t, src, dst, y2, 64, 32)

    w4p = jnp.zeros((64, 16), jnp.float32).at[:, :DIM].set(stn4_w)
    b4p = jnp.zeros((16,), jnp.float32).at[:DIM].set(stn4_b)
    t_pad = _tc_t(agg2, inv_deg, h1, stn2_root, stn2_b,
                  stn3_w, stn3_b, w4p, b4p)

    agg3 = _sc_gather_conv(pseudo_t, src, dst, y3, 128, 16, t_pad=t_pad)

    return _tc_final(agg3, inv_deg, input, conv_root, conv_b)
